# speedup vs baseline: 4.7105x; 4.7105x over previous
import functools

import jax
import jax.numpy as jnp
from jax import lax
from jax.experimental import pallas as pl
from jax.experimental.pallas import tpu as pltpu
from jax.experimental.pallas import tpu_sc as plsc

N = 10000
D = 128
E = 320000

NC = 2
NS = 16
NW = NC * NS

CHUNK = 128
CPT = -(-E // (CHUNK * NW))
PER_TILE = CPT * CHUNK
EPAD = PER_TILE * NW

NPAD = 10240
RPT = NPAD // NS
ZCH = RPT // CHUNK

_MESH = plsc.VectorSubcoreMesh(core_axis_name="c", subcore_axis_name="s")


@functools.partial(
    pl.kernel,
    out_type=jax.ShapeDtypeStruct((NC, NPAD, D), jnp.float32),
    mesh=_MESH,
    scratch_types=[
        pltpu.VMEM((CHUNK,), jnp.int32),
        pltpu.VMEM((CHUNK,), jnp.int32),
        pltpu.VMEM((CHUNK, D), jnp.float32),
        pltpu.VMEM((CHUNK, D), jnp.float32),
        pltpu.VMEM_SHARED((NPAD, D), jnp.float32),
        pltpu.SemaphoreType.DMA,
    ],
)
def _sc_aggregate(x_hbm, src_hbm, dst_hbm, zeros_hbm, out_hbm,
                  src_v, dst_v, rows_v, stage_v, acc_sh, sem):
    c = lax.axis_index("c")
    s = lax.axis_index("s")
    row0 = s * RPT

    pltpu.sync_copy(zeros_hbm, stage_v)

    def zero_body(i, carry):
        pltpu.sync_copy(stage_v, acc_sh.at[pl.ds(row0 + i * CHUNK, CHUNK), :])
        return carry

    lax.fori_loop(0, ZCH, zero_body, 0)
    plsc.subcore_barrier()

    base = (c * NS + s) * PER_TILE

    def edge_body(i, carry):
        off = base + i * CHUNK
        pltpu.sync_copy(src_hbm.at[pl.ds(off, CHUNK)], src_v)
        pltpu.sync_copy(dst_hbm.at[pl.ds(off, CHUNK)], dst_v)
        pltpu.async_copy(x_hbm.at[src_v], rows_v, sem).wait()
        pltpu.sync_copy(rows_v, acc_sh.at[dst_v], add=True)
        return carry

    lax.fori_loop(0, CPT, edge_body, 0)
    plsc.subcore_barrier()

    def drain_body(i, carry):
        r = row0 + i * CHUNK
        pltpu.sync_copy(acc_sh.at[pl.ds(r, CHUNK), :], stage_v)
        pltpu.sync_copy(stage_v, out_hbm.at[c, pl.ds(r, CHUNK), :])
        return carry

    lax.fori_loop(0, ZCH, drain_body, 0)


BM = 1024


def _merge_body(p_ref, w_ref, b_ref, o_ref):
    acc = p_ref[0] + p_ref[1]
    o_ref[...] = lax.dot_general(
        acc, w_ref[...], (((1,), (1,)), ((), ())),
        preferred_element_type=jnp.float32) + b_ref[...]


def _merge(partial, w, b2d):
    return pl.pallas_call(
        _merge_body,
        grid=(NPAD // BM,),
        in_specs=[
            pl.BlockSpec((NC, BM, D), lambda i: (0, i, 0)),
            pl.BlockSpec((D, D), lambda i: (0, 0)),
            pl.BlockSpec((1, D), lambda i: (0, 0)),
        ],
        out_specs=pl.BlockSpec((BM, D), lambda i: (i, 0)),
        out_shape=jax.ShapeDtypeStruct((NPAD, D), jnp.float32),
    )(partial, w, b2d)


def kernel(x, edge_index, W, b):
    src = edge_index[0].astype(jnp.int32)
    dst = edge_index[1].astype(jnp.int32)
    pad = EPAD - E
    src = jnp.concatenate([src, jnp.zeros((pad,), jnp.int32)])
    dst = jnp.concatenate([dst, jnp.full((pad,), N, jnp.int32)])
    zeros = jnp.zeros((CHUNK, D), jnp.float32)
    partial = _sc_aggregate(x, src, dst, zeros)
    out = _merge(partial, W, b.reshape(1, D))
    return out[:N]
